# baseline (speedup 1.0000x reference)
import numpy as np

import jax
import jax.numpy as jnp
from jax.experimental import pallas as pl
from jax.experimental.pallas import tpu as pltpu

D_MODEL = 1024
D_SPACE = 64
N_FQK = 2048
N_FV = 1024
N_REL = 2048
N_VAL = 1024
_POOLS = (
    (0, N_FQK, 64),
    (N_FQK, N_FV, 32),
    (N_FQK + N_FV, N_REL, 64),
    (N_FQK + N_FV + N_REL + N_VAL, N_REL, 64),
    (N_FQK + N_FV + N_REL, N_VAL, 32),
)
_N_TOTAL = N_FQK + N_FV + 2 * N_REL + N_VAL
_TB = 128


def _ce(v, i, d, s):
    N, TB = v.shape
    G = N // (2 * d)
    blk = jax.lax.broadcasted_iota(jnp.int32, (G, 1, 1), 0)
    desc = ((blk * (2 * d)) // s) % 2 == 0
    v4 = v.reshape(G, 2, d, TB)
    i4 = i.reshape(G, 2, d, TB)
    a, b = v4[:, 0], v4[:, 1]
    ai, bi = i4[:, 0], i4[:, 1]
    a_first = (a > b) | ((a == b) & (ai < bi))
    take_a = a_first == desc
    na = jnp.where(take_a, a, b)
    nb = jnp.where(take_a, b, a)
    nai = jnp.where(take_a, ai, bi)
    nbi = jnp.where(take_a, bi, ai)
    v = jnp.stack([na, nb], axis=1).reshape(N, TB)
    i = jnp.stack([nai, nbi], axis=1).reshape(N, TB)
    return v, i


def _topk_desc(v, k):
    N, TB = v.shape
    i = jax.lax.broadcasted_iota(jnp.int32, (N, TB), 0)
    s = 2
    while s <= k:
        d = s // 2
        while d >= 1:
            v, i = _ce(v, i, d, s)
            d //= 2
        s *= 2
    while v.shape[0] > k:
        N = v.shape[0]
        v3 = v.reshape(N // (2 * k), 2, k, TB)
        i3 = i.reshape(N // (2 * k), 2, k, TB)
        a, ai = v3[:, 0], i3[:, 0]
        b, bi = v3[:, 1], i3[:, 1]
        take_a = (a > b) | ((a == b) & (ai < bi))
        v = jnp.where(take_a, a, b).reshape(-1, TB)
        i = jnp.where(take_a, ai, bi).reshape(-1, TB)
        d = k // 2
        while d >= 1:
            v, i = _ce(v, i, d, k)
            d //= 2
    return v, i


def _router_body(x_ref, w_ref, b_ref, emb_ref, *out_refs):
    h = jnp.dot(x_ref[...], w_ref[...], preferred_element_type=jnp.float32)
    h = h + b_ref[...]
    for p, (start, n, k) in enumerate(_POOLS):
        e = emb_ref[start : start + n, :]
        inv = jax.lax.rsqrt(
            jnp.maximum(jnp.sum(e * e, axis=1, keepdims=True), 1e-24)
        )
        lg = jax.lax.dot_general(
            e * inv, h, (((1,), (1,)), ((), ())),
            preferred_element_type=jnp.float32,
        )
        vals, idx = _topk_desc(lg, k)
        p_exp = jnp.exp(vals - vals[0:1])
        w = p_exp / jnp.sum(p_exp, axis=0, keepdims=True)
        out_refs[2 * p][...] = w.T
        out_refs[2 * p + 1][...] = idx.T


def kernel(x, W_proj, b_proj, neuron_emb, neuron_emb_rk):
    B, S, _ = x.shape
    T = B * S
    xf = x.reshape(T, D_MODEL)
    emb = jnp.concatenate(
        [neuron_emb[: N_FQK + N_FV + N_REL + N_VAL], neuron_emb_rk], axis=0
    )
    out_shapes = []
    out_specs = []
    for (_, _, k) in _POOLS:
        out_shapes.append(jax.ShapeDtypeStruct((T, k), jnp.float32))
        out_shapes.append(jax.ShapeDtypeStruct((T, k), jnp.int32))
        out_specs.append(pl.BlockSpec((_TB, k), lambda i: (i, 0)))
        out_specs.append(pl.BlockSpec((_TB, k), lambda i: (i, 0)))
    outs = pl.pallas_call(
        _router_body,
        grid=(T // _TB,),
        in_specs=[
            pl.BlockSpec((_TB, D_MODEL), lambda i: (i, 0)),
            pl.BlockSpec((D_MODEL, D_SPACE), lambda i: (0, 0)),
            pl.BlockSpec((1, D_SPACE), lambda i: (0, 0)),
            pl.BlockSpec((_N_TOTAL, D_SPACE), lambda i: (0, 0)),
        ],
        out_specs=out_specs,
        out_shape=out_shapes,
    )(xf, W_proj, b_proj.reshape(1, D_SPACE), emb)
    result = []
    for p, (_, _, k) in enumerate(_POOLS):
        result.append(outs[2 * p].reshape(B, S, k))
        result.append(outs[2 * p + 1].reshape(B, S, k))
    return tuple(result)


# speedup vs baseline: 12.7817x; 12.7817x over previous
import jax
import jax.numpy as jnp
from jax.experimental import pallas as pl

D_MODEL = 1024
D_SPACE = 64
N_FQK = 2048
N_FV = 1024
N_REL = 2048
N_VAL = 1024
_S_FQK = 0
_S_FV = N_FQK
_S_RQ = N_FQK + N_FV
_S_V = N_FQK + N_FV + N_REL
_S_RK = N_FQK + N_FV + N_REL + N_VAL
_N_TOTAL = N_FQK + N_FV + 2 * N_REL + N_VAL
_TB = 128


def _ce_strided(v, i, d, G, s, half_dir):
    N, TB = v.shape
    dG = d * G
    B2 = N // (2 * dG)
    v4 = v.reshape(B2, 2, dG, TB)
    i4 = i.reshape(B2, 2, dG, TB)
    a, b = v4[:, 0], v4[:, 1]
    ai, bi = i4[:, 0], i4[:, 1]
    blk = jax.lax.broadcasted_iota(jnp.int32, (B2, 1, 1), 0)
    stage_desc = ((blk * (2 * d)) // s) % 2 == 0
    if half_dir:
        pos = jax.lax.broadcasted_iota(jnp.int32, (1, dG, 1), 1)
        desc = stage_desc != ((pos % G) >= (G // 2))
    else:
        desc = stage_desc
    take_a = (a >= b) == desc
    na = jnp.where(take_a, a, b)
    nb = jnp.where(take_a, b, a)
    nai = jnp.where(take_a, ai, bi)
    nbi = jnp.where(take_a, bi, ai)
    v = jnp.stack([na, nb], axis=1).reshape(N, TB)
    i = jnp.stack([nai, nbi], axis=1).reshape(N, TB)
    return v, i


def _topk_desc(v, k):
    N, TB = v.shape
    G = N // k
    i = jax.lax.broadcasted_iota(jnp.int32, (N, TB), 0)
    s = 2
    while s <= k:
        d = s // 2
        while d >= 1:
            v, i = _ce_strided(v, i, d, G, s, half_dir=(G > 1))
            d //= 2
        s *= 2
    while G > 1:
        Gh = G // 2
        v3 = v.reshape(k, G, TB)
        i3 = i.reshape(k, G, TB)
        a, b = v3[:, :Gh], v3[:, Gh:]
        ai, bi = i3[:, :Gh], i3[:, Gh:]
        take_a = a >= b
        v = jnp.where(take_a, a, b).reshape(k * Gh, TB)
        i = jnp.where(take_a, ai, bi).reshape(k * Gh, TB)
        d = k // 2
        while d >= 1:
            v, i = _ce_strided(v, i, d, Gh, 2 * k, half_dir=(Gh > 1))
            d //= 2
        G = Gh
    return v, i


def _finish(vals, idx):
    p = jnp.exp(vals - vals[0:1])
    w = p / jnp.sum(p, axis=0, keepdims=True)
    return w.T, idx.T


def _router_body(x_ref, w_ref, b_ref, emb_ref,
                 fqk_w, fqk_i, fv_w, fv_i, rq_w, rq_i, rk_w, rk_i, v_w, v_i):
    h = jnp.dot(x_ref[...], w_ref[...], preferred_element_type=jnp.float32)
    h = h + b_ref[...]

    def pool_logits(start, n):
        e = emb_ref[start : start + n, :]
        return jax.lax.dot_general(
            e, h, (((1,), (1,)), ((), ())), preferred_element_type=jnp.float32
        )

    lg_a = jnp.concatenate(
        [pool_logits(_S_FQK, N_FQK), pool_logits(_S_RQ, N_REL),
         pool_logits(_S_RK, N_REL)], axis=1)
    va, ia = _topk_desc(lg_a, 64)
    wa, ia_t = _finish(va, ia)
    fqk_w[...], fqk_i[...] = wa[:_TB], ia_t[:_TB]
    rq_w[...], rq_i[...] = wa[_TB : 2 * _TB], ia_t[_TB : 2 * _TB]
    rk_w[...], rk_i[...] = wa[2 * _TB :], ia_t[2 * _TB :]

    lg_b = jnp.concatenate(
        [pool_logits(_S_FV, N_FV), pool_logits(_S_V, N_VAL)], axis=1)
    vb, ib = _topk_desc(lg_b, 32)
    wb, ib_t = _finish(vb, ib)
    fv_w[...], fv_i[...] = wb[:_TB], ib_t[:_TB]
    v_w[...], v_i[...] = wb[_TB:], ib_t[_TB:]


def kernel(x, W_proj, b_proj, neuron_emb, neuron_emb_rk):
    B, S, _ = x.shape
    T = B * S
    xf = x.reshape(T, D_MODEL)
    emb = jnp.concatenate(
        [neuron_emb[: N_FQK + N_FV + N_REL + N_VAL], neuron_emb_rk], axis=0
    )
    emb = emb / jnp.clip(jnp.linalg.norm(emb, axis=-1, keepdims=True), 1e-12)
    ks = (64, 32, 64, 64, 32)
    out_shapes = []
    out_specs = []
    for k in ks:
        out_shapes.append(jax.ShapeDtypeStruct((T, k), jnp.float32))
        out_shapes.append(jax.ShapeDtypeStruct((T, k), jnp.int32))
        out_specs.append(pl.BlockSpec((_TB, k), lambda i: (i, 0)))
        out_specs.append(pl.BlockSpec((_TB, k), lambda i: (i, 0)))
    outs = pl.pallas_call(
        _router_body,
        grid=(T // _TB,),
        in_specs=[
            pl.BlockSpec((_TB, D_MODEL), lambda i: (i, 0)),
            pl.BlockSpec((D_MODEL, D_SPACE), lambda i: (0, 0)),
            pl.BlockSpec((1, D_SPACE), lambda i: (0, 0)),
            pl.BlockSpec((_N_TOTAL, D_SPACE), lambda i: (0, 0)),
        ],
        out_specs=out_specs,
        out_shape=out_shapes,
    )(xf, W_proj, b_proj.reshape(1, D_SPACE), emb)
    result = []
    for p, k in enumerate(ks):
        result.append(outs[2 * p].reshape(B, S, k))
        result.append(outs[2 * p + 1].reshape(B, S, k))
    return tuple(result)
